# baseline (speedup 1.0000x reference)
import functools
import jax
import jax.numpy as jnp
import numpy as np
from jax import lax
from jax.experimental import pallas as pl
from jax.experimental.pallas import tpu as pltpu
from jax.experimental.pallas import tpu_sc as plsc

B = 16
N = 2048
M = 512
K = 32
CF = 61
CIN = 64
RAD = np.float32(0.2)
THR2 = np.float32(0.2 ** 2)
NPOS = B * M * K
EPS = np.float32(1e-5)


def _fps_body(pts_ref, cent_ref):
    px = pts_ref[0]
    py = pts_ref[1]
    pz = pts_ref[2]
    iota = lax.broadcasted_iota(jnp.int32, (B, N), 1)
    iota_m = lax.broadcasted_iota(jnp.int32, (B, M), 1)
    dis0 = jnp.full((B, N), 1e10, dtype=jnp.float32)
    far0 = jnp.zeros((B, 1), dtype=jnp.int32)
    cz0 = jnp.zeros((B, M), dtype=jnp.float32)

    def body(i, st):
        dis, far, ox, oy, oz = st
        oh = iota == far
        cx = jnp.sum(jnp.where(oh, px, 0.0), axis=1, keepdims=True)
        cy = jnp.sum(jnp.where(oh, py, 0.0), axis=1, keepdims=True)
        cz = jnp.sum(jnp.where(oh, pz, 0.0), axis=1, keepdims=True)
        slot = iota_m == i
        ox = jnp.where(slot, cx, ox)
        oy = jnp.where(slot, cy, oy)
        oz = jnp.where(slot, cz, oz)
        dx = px - cx
        dy = py - cy
        dz = pz - cz
        d = dx * dx
        d = d + dy * dy
        d = d + dz * dz
        dis = jnp.minimum(dis, d)
        mx = jnp.max(dis, axis=1, keepdims=True)
        cand = jnp.where(dis == mx, iota, N)
        far = jnp.min(cand, axis=1, keepdims=True)
        return (dis, far, ox, oy, oz)

    _, _, ox, oy, oz = lax.fori_loop(0, M, body, (dis0, far0, cz0, cz0, cz0))
    cent_ref[0] = ox
    cent_ref[1] = oy
    cent_ref[2] = oz


def _fps(pts_t):
    return pl.pallas_call(
        _fps_body,
        out_shape=jax.ShapeDtypeStruct((3, B, M), jnp.float32),
    )(pts_t)


def _v_body(p_ref, f_ref, w0pt_ref, w0ft_ref, v_ref):
    v_ref[0] = (
        jnp.dot(p_ref[0] / RAD, w0pt_ref[...],
                preferred_element_type=jnp.float32)
        + jnp.dot(f_ref[0], w0ft_ref[...],
                  preferred_element_type=jnp.float32))


def _v_mat(points, features, w0pt, w0ft):
    return pl.pallas_call(
        _v_body,
        grid=(B,),
        in_specs=[
            pl.BlockSpec((1, N, 3), lambda b: (b, 0, 0)),
            pl.BlockSpec((1, N, CF), lambda b: (b, 0, 0)),
            pl.BlockSpec((3, CIN), lambda b: (0, 0)),
            pl.BlockSpec((CF, CIN), lambda b: (0, 0)),
        ],
        out_specs=pl.BlockSpec((1, N, CIN), lambda b: (b, 0, 0)),
        out_shape=jax.ShapeDtypeStruct((B, N, CIN), jnp.float32),
    )(points, features, w0pt, w0ft)


def _beff_body(cr_ref, w0pt_ref, b0_ref, out_ref):
    out_ref[...] = b0_ref[...] - jnp.dot(
        cr_ref[...], w0pt_ref[...], preferred_element_type=jnp.float32)


def _beff(c_flat_r, w0pt, b0row):
    return pl.pallas_call(
        _beff_body,
        out_shape=jax.ShapeDtypeStruct((B * M, 128), jnp.float32),
    )(c_flat_r, w0pt, b0row)


_NSUB = N // 128


def _sqr_body(cent_ref, pts_ref, out_ref):
    cx = cent_ref[0, 0].reshape(M, 1)
    cy = cent_ref[0, 1].reshape(M, 1)
    cz = cent_ref[0, 2].reshape(M, 1)
    px = pts_ref[0, 0].reshape(1, N)
    py = pts_ref[0, 1].reshape(1, N)
    pz = pts_ref[0, 2].reshape(1, N)
    dx = cx - px
    dy = cy - py
    dz = cz - pz
    d = dx * dx
    d = d + dy * dy
    d = d + dz * dz
    out_ref[0] = d


def _sqr(cent_b, pts_b):
    return pl.pallas_call(
        _sqr_body,
        grid=(B,),
        in_specs=[
            pl.BlockSpec((1, 3, M), lambda b: (b, 0, 0)),
            pl.BlockSpec((1, 3, N), lambda b: (b, 0, 0)),
        ],
        out_specs=pl.BlockSpec((1, M, N), lambda b: (b, 0, 0)),
        out_shape=jax.ShapeDtypeStruct((B, M, N), jnp.float32),
    )(cent_b, pts_b)


_NWORK = 32
_CPW = (B * M) // _NWORK
_NCH = N // 16


_RG = 4
_NG = _CPW // _RG
_GR = _RG * K
_DEPTH = 4


def _sc_body(sqr_hbm, v_hbm, out_hbm, bufs, idxbufs, fidx, grows,
             isem, gsem, wsem):
    cid = lax.axis_index("c")
    sid = lax.axis_index("s")
    wid = sid * 2 + cid
    row0 = wid * _CPW
    base = (row0 // M) * N
    iota16 = lax.iota(jnp.int32, 16)
    big0 = jnp.full((16,), 2**30, jnp.int32)

    def gout_slice(g):
        return out_hbm.at[pl.dslice((row0 + _RG * g) * K, _GR)]

    for q in range(_DEPTH):
        pltpu.async_copy(sqr_hbm.at[pl.dslice(row0 + _RG * q, _RG)],
                         bufs.at[q], isem)

    def super_iter(s, _):
        for p in range(_DEPTH):
            g = _DEPTH * s + p
            bufp = bufs.at[p]
            pltpu.make_async_copy(
                sqr_hbm.at[pl.dslice(0, _RG)], bufp, isem).wait()

            def scan(ch0, ws):
                gi0 = base + ch0 * 128
                new_ws = list(ws)
                for ch1 in range(8):
                    gi = (gi0 + ch1 * 16) + iota16
                    for r in range(_RG):
                        v = bufp[r, pl.dslice(ch0 * 128 + ch1 * 16, 16)]
                        msk = v <= THR2
                        pc = plsc.cumsum(msk.astype(jnp.int32))
                        pos = new_ws[r] + pc
                        plsc.store_scatter(idxbufs.at[r], [pos], gi, mask=msk)
                        new_ws[r] = new_ws[r] + pc[15]
                return tuple(new_ws)

            init = (jnp.int32(0),) * _RG
            st = lax.fori_loop(0, _NSUB, scan, init)
            for r in range(_RG):
                total = st[r]
                head = idxbufs[r, pl.dslice(1, 16)]
                first = head[0]
                for j in range(2):
                    valid = (j * 16 + iota16) < total
                    vals = idxbufs[r, pl.dslice(1 + j * 16, 16)]
                    fidx[p, pl.dslice(r * K + j * 16, 16)] = \
                        jnp.where(valid, vals, first)

            pm2 = (p - 2) % _DEPTH

            @pl.when(g >= 2)
            def _():
                pltpu.make_async_copy(
                    v_hbm.at[fidx.at[pm2]], grows.at[pm2], gsem).wait()
                pltpu.async_copy(grows.at[pm2], gout_slice(g - 2), wsem)

            @pl.when(g >= _DEPTH)
            def _():
                pltpu.make_async_copy(
                    grows.at[p], gout_slice(0), wsem).wait()

            pltpu.async_copy(v_hbm.at[fidx.at[p]], grows.at[p], gsem)

            @pl.when(g + _DEPTH < _NG)
            def _():
                pltpu.async_copy(
                    sqr_hbm.at[pl.dslice(row0 + _RG * (g + _DEPTH), _RG)],
                    bufp, isem)
        return 0

    lax.fori_loop(0, _NG // _DEPTH, super_iter, 0)
    for g in (_NG - 2, _NG - 1):
        q = g % _DEPTH
        pltpu.make_async_copy(v_hbm.at[fidx.at[q]], grows.at[q], gsem).wait()
        pltpu.async_copy(grows.at[q], gout_slice(g), wsem)
    for q in range(_DEPTH):
        pltpu.make_async_copy(grows.at[q], gout_slice(0), wsem).wait()


def _sc_gather(sqr_flat, v_flat):
    mesh = plsc.VectorSubcoreMesh(core_axis_name="c", subcore_axis_name="s",
                                  num_cores=2, num_subcores=16)
    fn = pl.kernel(
        _sc_body,
        out_type=jax.ShapeDtypeStruct((NPOS, CIN), jnp.float32),
        mesh=mesh,
        compiler_params=pltpu.CompilerParams(
            needs_layout_passes=False, use_tc_tiling_on_sc=False),
        scratch_types=[
            pltpu.VMEM((_DEPTH, _RG, N), jnp.float32),
            pltpu.VMEM((_RG, N + 48), jnp.int32),
            pltpu.VMEM((_DEPTH, _GR), jnp.int32),
            pltpu.VMEM((_DEPTH, _GR, CIN), jnp.float32),
            pltpu.SemaphoreType.DMA,
            pltpu.SemaphoreType.DMA,
            pltpu.SemaphoreType.DMA,
        ],
    )
    return fn(sqr_flat, v_flat)


_PR = NPOS // 2
_RB = 4096
_NB = _PR // _RB
_CB = _RB // (K // 2)
_PK = K // 2
_W = 128
_COUT = 128


def _stats_body(x_ref, beff_ref, out_ref):
    i = pl.program_id(0)
    x = x_ref[...].reshape(_CB, _PK, _W) + beff_ref[...].reshape(_CB, 1, _W)
    s = jnp.sum(x, axis=(0, 1))
    q = jnp.sum(x * x, axis=(0, 1))
    sq = jnp.stack([s, q], axis=0)

    @pl.when(i == 0)
    def _():
        out_ref[...] = sq

    @pl.when(i != 0)
    def _():
        out_ref[...] += sq


def _stats1(xgp, beff):
    return pl.pallas_call(
        _stats_body,
        grid=(_NB,),
        in_specs=[
            pl.BlockSpec((_RB, _W), lambda i: (i, 0)),
            pl.BlockSpec((_CB, _W), lambda i: (i, 0)),
        ],
        out_specs=pl.BlockSpec((2, _W), lambda i: (0, 0)),
        out_shape=jax.ShapeDtypeStruct((2, _W), jnp.float32),
    )(xgp, beff)


def _norm_coefs(stats, g, be, w):
    s = stats[0:1, :w] + stats[0:1, w:]
    q = stats[1:2, :w] + stats[1:2, w:]
    mean = s * np.float32(1.0 / NPOS)
    var = q * np.float32(1.0 / NPOS) - mean * mean
    a = g * lax.rsqrt(var + EPS)
    c = be - a * mean
    a2 = jnp.concatenate([a, a], axis=1)
    c2 = jnp.concatenate([c, c], axis=1)
    return a2, c2


def _layer2_body(x_ref, beff_ref, st_ref, w1bd_ref, g0_ref, be0_ref, b1_ref,
                 y_ref, out_st_ref):
    i = pl.program_id(0)
    a, c = _norm_coefs(st_ref[...], g0_ref[...], be0_ref[...], CIN)
    x = x_ref[...].reshape(_CB, _PK, _W) + beff_ref[...].reshape(_CB, 1, _W)
    h = jnp.maximum(a.reshape(1, 1, _W) * x + c.reshape(1, 1, _W), 0.0)
    y = jnp.dot(h.reshape(_RB, _W), w1bd_ref[...],
                preferred_element_type=jnp.float32) + b1_ref[...]
    y_ref[...] = y
    s = jnp.sum(y, axis=0)
    q = jnp.sum(y * y, axis=0)
    sq = jnp.stack([s, q], axis=0)

    @pl.when(i == 0)
    def _():
        out_st_ref[...] = sq

    @pl.when(i != 0)
    def _():
        out_st_ref[...] += sq


def _layer2(xgp, beff, st1, w1bd, g0, be0, b1p):
    return pl.pallas_call(
        _layer2_body,
        grid=(_NB,),
        in_specs=[
            pl.BlockSpec((_RB, _W), lambda i: (i, 0)),
            pl.BlockSpec((_CB, _W), lambda i: (i, 0)),
            pl.BlockSpec((2, _W), lambda i: (0, 0)),
            pl.BlockSpec((_W, _W), lambda i: (0, 0)),
            pl.BlockSpec((1, CIN), lambda i: (0, 0)),
            pl.BlockSpec((1, CIN), lambda i: (0, 0)),
            pl.BlockSpec((1, _W), lambda i: (0, 0)),
        ],
        out_specs=[
            pl.BlockSpec((_RB, _W), lambda i: (i, 0)),
            pl.BlockSpec((2, _W), lambda i: (0, 0)),
        ],
        out_shape=[
            jax.ShapeDtypeStruct((_PR, _W), jnp.float32),
            jax.ShapeDtypeStruct((2, _W), jnp.float32),
        ],
    )(xgp, beff, st1, w1bd, g0, be0, b1p)


def _layer3_body(y_ref, st_ref, w2bd_ref, g1_ref, be1_ref, b2p_ref,
                 m_ref, out_st_ref):
    i = pl.program_id(0)
    a, c = _norm_coefs(st_ref[...], g1_ref[...], be1_ref[...], CIN)
    h = jnp.maximum(a * y_ref[...] + c, 0.0)
    y = jnp.dot(h, w2bd_ref[...],
                preferred_element_type=jnp.float32) + b2p_ref[...]
    s = jnp.sum(y, axis=0)
    q = jnp.sum(y * y, axis=0)
    sq = jnp.stack([s, q], axis=0)
    m_ref[...] = jnp.max(y.reshape(_CB, _PK, 2, _COUT), axis=(1, 2))

    @pl.when(i == 0)
    def _():
        out_st_ref[...] = sq

    @pl.when(i != 0)
    def _():
        out_st_ref[...] += sq


def _layer3(y2, st2, w2bd, g1, be1, b2p):
    return pl.pallas_call(
        _layer3_body,
        grid=(_NB,),
        in_specs=[
            pl.BlockSpec((_RB, _W), lambda i: (i, 0)),
            pl.BlockSpec((2, _W), lambda i: (0, 0)),
            pl.BlockSpec((_W, 2 * _COUT), lambda i: (0, 0)),
            pl.BlockSpec((1, CIN), lambda i: (0, 0)),
            pl.BlockSpec((1, CIN), lambda i: (0, 0)),
            pl.BlockSpec((1, 2 * _COUT), lambda i: (0, 0)),
        ],
        out_specs=[
            pl.BlockSpec((_CB, _COUT), lambda i: (i, 0)),
            pl.BlockSpec((2, 2 * _COUT), lambda i: (0, 0)),
        ],
        out_shape=[
            jax.ShapeDtypeStruct((B * M, _COUT), jnp.float32),
            jax.ShapeDtypeStruct((2, 2 * _COUT), jnp.float32),
        ],
    )(y2, st2, w2bd, g1, be1, b2p)


def _final_body(m_ref, st_ref, g2_ref, be2_ref, out_ref):
    a, c = _norm_coefs(st_ref[...], g2_ref[...], be2_ref[...], _COUT)
    out_ref[...] = jnp.maximum(a[:, :_COUT] * m_ref[...] + c[:, :_COUT], 0.0)


def _final(m3, st3, g2, be2):
    return pl.pallas_call(
        _final_body,
        out_shape=jax.ShapeDtypeStruct((B * M, _COUT), jnp.float32),
    )(m3, st3, g2, be2)


@jax.jit
def kernel(points, features, W0, b0, g0, be0, W1, b1, g1, be1,
           W2, b2, g2, be2):
    pts_t = points.transpose(2, 0, 1)
    cent_t = _fps(pts_t)
    centroids = cent_t.transpose(1, 2, 0)

    v = _v_mat(points, features, W0[:, :3].T, W0[:, 3:].T)
    w0pt2 = jnp.concatenate([W0[:, :3].T, W0[:, :3].T], axis=1)
    b0p = jnp.concatenate([b0, b0]).reshape(1, _W)
    beff = _beff((centroids / RAD).reshape(B * M, 3), w0pt2, b0p)
    sqr = _sqr(cent_t.transpose(1, 0, 2), pts_t.transpose(1, 0, 2))

    xg = _sc_gather(sqr.reshape(B * M, N), v.reshape(B * N, CIN))
    xgp = xg.reshape(_PR, _W)

    w1bd = jnp.zeros((_W, _W), jnp.float32)
    w1bd = w1bd.at[:CIN, :CIN].set(W1.T).at[CIN:, CIN:].set(W1.T)
    w2bd = jnp.zeros((_W, 2 * _COUT), jnp.float32)
    w2bd = w2bd.at[:CIN, :_COUT].set(W2.T).at[CIN:, _COUT:].set(W2.T)
    b1p = jnp.concatenate([b1, b1]).reshape(1, _W)
    b2p = jnp.concatenate([b2, b2]).reshape(1, 2 * _COUT)

    st1 = _stats1(xgp, beff)
    y2, st2 = _layer2(xgp, beff, st1, w1bd, g0.reshape(1, CIN),
                      be0.reshape(1, CIN), b1p)
    m3, st3 = _layer3(y2, st2, w2bd, g1.reshape(1, CIN),
                      be1.reshape(1, CIN), b2p)
    g = _final(m3, st3, g2.reshape(1, _COUT), be2.reshape(1, _COUT))
    return (centroids, g.reshape(B, M, _COUT))


# speedup vs baseline: 1.0039x; 1.0039x over previous
import jax
import jax.numpy as jnp
import numpy as np
from jax import lax
from jax.experimental import pallas as pl
from jax.experimental.pallas import tpu as pltpu
from jax.experimental.pallas import tpu_sc as plsc

B = 16
N = 2048
M = 512
K = 32
CF = 61
CIN = 64
RAD = np.float32(0.2)
THR2 = np.float32(0.2 ** 2)
NPOS = B * M * K
EPS = np.float32(1e-5)


def _fps_body(pts_ref, cent_ref):
    px = pts_ref[0]
    py = pts_ref[1]
    pz = pts_ref[2]
    iota = lax.broadcasted_iota(jnp.int32, (B, N), 1)
    iota_m = lax.broadcasted_iota(jnp.int32, (B, M), 1)
    dis0 = jnp.full((B, N), 1e10, dtype=jnp.float32)
    far0 = jnp.zeros((B, 1), dtype=jnp.int32)
    cz0 = jnp.zeros((B, M), dtype=jnp.float32)

    def body(i, st):
        dis, far, ox, oy, oz = st
        oh = iota == far
        cx = jnp.sum(jnp.where(oh, px, 0.0), axis=1, keepdims=True)
        cy = jnp.sum(jnp.where(oh, py, 0.0), axis=1, keepdims=True)
        cz = jnp.sum(jnp.where(oh, pz, 0.0), axis=1, keepdims=True)
        slot = iota_m == i
        ox = jnp.where(slot, cx, ox)
        oy = jnp.where(slot, cy, oy)
        oz = jnp.where(slot, cz, oz)
        dx = px - cx
        dy = py - cy
        dz = pz - cz
        d = dx * dx
        d = d + dy * dy
        d = d + dz * dz
        dis = jnp.minimum(dis, d)
        mx = jnp.max(dis, axis=1, keepdims=True)
        cand = jnp.where(dis == mx, iota, N)
        far = jnp.min(cand, axis=1, keepdims=True)
        return (dis, far, ox, oy, oz)

    _, _, ox, oy, oz = lax.fori_loop(0, M, body, (dis0, far0, cz0, cz0, cz0))
    cent_ref[0] = ox
    cent_ref[1] = oy
    cent_ref[2] = oz


def _fps(pts_t):
    return pl.pallas_call(
        _fps_body,
        out_shape=jax.ShapeDtypeStruct((3, B, M), jnp.float32),
    )(pts_t)


def _v_body(p_ref, f_ref, w0pt_ref, w0ft_ref, v_ref):
    v_ref[0] = (
        jnp.dot(p_ref[0] / RAD, w0pt_ref[...],
                preferred_element_type=jnp.float32)
        + jnp.dot(f_ref[0], w0ft_ref[...],
                  preferred_element_type=jnp.float32))


def _v_mat(points, features, w0pt, w0ft):
    return pl.pallas_call(
        _v_body,
        grid=(B,),
        in_specs=[
            pl.BlockSpec((1, N, 3), lambda b: (b, 0, 0)),
            pl.BlockSpec((1, N, CF), lambda b: (b, 0, 0)),
            pl.BlockSpec((3, CIN), lambda b: (0, 0)),
            pl.BlockSpec((CF, CIN), lambda b: (0, 0)),
        ],
        out_specs=pl.BlockSpec((1, N, CIN), lambda b: (b, 0, 0)),
        out_shape=jax.ShapeDtypeStruct((B, N, CIN), jnp.float32),
    )(points, features, w0pt, w0ft)


def _beff_body(cr_ref, w0pt_ref, b0_ref, out_ref):
    out_ref[...] = b0_ref[...] - jnp.dot(
        cr_ref[...], w0pt_ref[...], preferred_element_type=jnp.float32)


def _beff(c_flat_r, w0pt, b0row):
    return pl.pallas_call(
        _beff_body,
        out_shape=jax.ShapeDtypeStruct((B * M, 128), jnp.float32),
    )(c_flat_r, w0pt, b0row)


_NSUB = N // 128


def _sqr_body(cent_ref, pts_ref, out_ref):
    cx = cent_ref[0, 0].reshape(M, 1)
    cy = cent_ref[0, 1].reshape(M, 1)
    cz = cent_ref[0, 2].reshape(M, 1)
    px = pts_ref[0, 0].reshape(1, N)
    py = pts_ref[0, 1].reshape(1, N)
    pz = pts_ref[0, 2].reshape(1, N)
    dx = cx - px
    dy = cy - py
    dz = cz - pz
    d = dx * dx
    d = d + dy * dy
    d = d + dz * dz
    out_ref[0] = d


def _sqr(cent_b, pts_b):
    return pl.pallas_call(
        _sqr_body,
        grid=(B,),
        in_specs=[
            pl.BlockSpec((1, 3, M), lambda b: (b, 0, 0)),
            pl.BlockSpec((1, 3, N), lambda b: (b, 0, 0)),
        ],
        out_specs=pl.BlockSpec((1, M, N), lambda b: (b, 0, 0)),
        out_shape=jax.ShapeDtypeStruct((B, M, N), jnp.float32),
    )(cent_b, pts_b)


_NWORK = 32
_CPW = (B * M) // _NWORK


_RG = 8
_NG = _CPW // _RG
_GR = _RG * K
_DEPTH = 2


def _sc_body(sqr_hbm, v_hbm, out_hbm, bufs, idxbufs, fidx, grows,
             isem, gsem, wsem):
    cid = lax.axis_index("c")
    sid = lax.axis_index("s")
    wid = sid * 2 + cid
    row0 = wid * _CPW
    base = (row0 // M) * N
    iota16 = lax.iota(jnp.int32, 16)
    big0 = jnp.full((16,), 2**30, jnp.int32)

    def gout_slice(g):
        return out_hbm.at[pl.dslice((row0 + _RG * g) * K, _GR)]

    for q in range(_DEPTH):
        pltpu.async_copy(sqr_hbm.at[pl.dslice(row0 + _RG * q, _RG)],
                         bufs.at[q], isem)

    def super_iter(s, _):
        for p in range(_DEPTH):
            g = _DEPTH * s + p
            bufp = bufs.at[p]
            pltpu.make_async_copy(
                sqr_hbm.at[pl.dslice(0, _RG)], bufp, isem).wait()

            def scan(ch0, ws):
                gi0 = base + ch0 * 128
                new_ws = list(ws)
                for ch1 in range(8):
                    gi = (gi0 + ch1 * 16) + iota16
                    for r in range(_RG):
                        v = bufp[r, pl.dslice(ch0 * 128 + ch1 * 16, 16)]
                        msk = v <= THR2
                        pc = plsc.cumsum(msk.astype(jnp.int32))
                        pos = new_ws[r] + pc
                        plsc.store_scatter(idxbufs.at[r], [pos], gi, mask=msk)
                        new_ws[r] = new_ws[r] + pc[15]
                return tuple(new_ws)

            init = (jnp.int32(0),) * _RG
            st = lax.fori_loop(0, _NSUB, scan, init)
            for r in range(_RG):
                total = st[r]
                head = idxbufs[r, pl.dslice(1, 16)]
                first = head[0]
                for j in range(2):
                    off = r * K + j * 16
                    valid = (j * 16 + iota16) < total
                    vals = idxbufs[r, pl.dslice(1 + j * 16, 16)]
                    fidx[p, off // 128, pl.dslice(off % 128, 16)] = \
                        jnp.where(valid, vals, first)

            pm1 = (p - 1) % _DEPTH

            @pl.when(g >= 1)
            def _():
                for hh in range(2):
                    pltpu.make_async_copy(
                        v_hbm.at[fidx.at[pm1, hh]],
                        grows.at[pm1, pl.dslice(hh * 128, 128)], gsem).wait()
                pltpu.async_copy(grows.at[pm1], gout_slice(g - 1), wsem)

            @pl.when(g >= _DEPTH)
            def _():
                pltpu.make_async_copy(
                    grows.at[p], gout_slice(0), wsem).wait()

            for hh in range(2):
                pltpu.async_copy(v_hbm.at[fidx.at[p, hh]],
                                 grows.at[p, pl.dslice(hh * 128, 128)], gsem)

            @pl.when(g + _DEPTH < _NG)
            def _():
                pltpu.async_copy(
                    sqr_hbm.at[pl.dslice(row0 + _RG * (g + _DEPTH), _RG)],
                    bufp, isem)
        return 0

    lax.fori_loop(0, _NG // _DEPTH, super_iter, 0)
    last = _NG - 1
    q = last % _DEPTH
    for hh in range(2):
        pltpu.make_async_copy(v_hbm.at[fidx.at[q, hh]],
                              grows.at[q, pl.dslice(hh * 128, 128)],
                              gsem).wait()
    pltpu.async_copy(grows.at[q], gout_slice(last), wsem)
    for q2 in range(_DEPTH):
        pltpu.make_async_copy(grows.at[q2], gout_slice(0), wsem).wait()


def _sc_gather(sqr_flat, v_flat):
    mesh = plsc.VectorSubcoreMesh(core_axis_name="c", subcore_axis_name="s",
                                  num_cores=2, num_subcores=16)
    fn = pl.kernel(
        _sc_body,
        out_type=jax.ShapeDtypeStruct((NPOS, CIN), jnp.float32),
        mesh=mesh,
        compiler_params=pltpu.CompilerParams(
            needs_layout_passes=False, use_tc_tiling_on_sc=False),
        scratch_types=[
            pltpu.VMEM((_DEPTH, _RG, N), jnp.float32),
            pltpu.VMEM((_RG, N + 48), jnp.int32),
            pltpu.VMEM((_DEPTH, 2, 128), jnp.int32),
            pltpu.VMEM((_DEPTH, _GR, CIN), jnp.float32),
            pltpu.SemaphoreType.DMA,
            pltpu.SemaphoreType.DMA,
            pltpu.SemaphoreType.DMA,
        ],
    )
    return fn(sqr_flat, v_flat)


_PR = NPOS // 2
_RB = 4096
_NB = _PR // _RB
_CB = _RB // (K // 2)
_PK = K // 2
_W = 128
_COUT = 128


def _stats_body(x_ref, beff_ref, out_ref):
    i = pl.program_id(0)
    x = x_ref[...].reshape(_CB, _PK, _W) + beff_ref[...].reshape(_CB, 1, _W)
    s = jnp.sum(x, axis=(0, 1))
    q = jnp.sum(x * x, axis=(0, 1))
    sq = jnp.stack([s, q], axis=0)

    @pl.when(i == 0)
    def _():
        out_ref[...] = sq

    @pl.when(i != 0)
    def _():
        out_ref[...] += sq


def _stats1(xgp, beff):
    return pl.pallas_call(
        _stats_body,
        grid=(_NB,),
        in_specs=[
            pl.BlockSpec((_RB, _W), lambda i: (i, 0)),
            pl.BlockSpec((_CB, _W), lambda i: (i, 0)),
        ],
        out_specs=pl.BlockSpec((2, _W), lambda i: (0, 0)),
        out_shape=jax.ShapeDtypeStruct((2, _W), jnp.float32),
    )(xgp, beff)


def _norm_coefs(stats, g, be, w):
    s = stats[0:1, :w] + stats[0:1, w:]
    q = stats[1:2, :w] + stats[1:2, w:]
    mean = s * np.float32(1.0 / NPOS)
    var = q * np.float32(1.0 / NPOS) - mean * mean
    a = g * lax.rsqrt(var + EPS)
    c = be - a * mean
    a2 = jnp.concatenate([a, a], axis=1)
    c2 = jnp.concatenate([c, c], axis=1)
    return a2, c2


def _layer2_body(x_ref, beff_ref, st_ref, w1bd_ref, g0_ref, be0_ref, b1_ref,
                 y_ref, out_st_ref):
    i = pl.program_id(0)
    a, c = _norm_coefs(st_ref[...], g0_ref[...], be0_ref[...], CIN)
    x = x_ref[...].reshape(_CB, _PK, _W) + beff_ref[...].reshape(_CB, 1, _W)
    h = jnp.maximum(a.reshape(1, 1, _W) * x + c.reshape(1, 1, _W), 0.0)
    y = jnp.dot(h.reshape(_RB, _W), w1bd_ref[...],
                preferred_element_type=jnp.float32) + b1_ref[...]
    y_ref[...] = y
    s = jnp.sum(y, axis=0)
    q = jnp.sum(y * y, axis=0)
    sq = jnp.stack([s, q], axis=0)

    @pl.when(i == 0)
    def _():
        out_st_ref[...] = sq

    @pl.when(i != 0)
    def _():
        out_st_ref[...] += sq


def _layer2(xgp, beff, st1, w1bd, g0, be0, b1p):
    return pl.pallas_call(
        _layer2_body,
        grid=(_NB,),
        in_specs=[
            pl.BlockSpec((_RB, _W), lambda i: (i, 0)),
            pl.BlockSpec((_CB, _W), lambda i: (i, 0)),
            pl.BlockSpec((2, _W), lambda i: (0, 0)),
            pl.BlockSpec((_W, _W), lambda i: (0, 0)),
            pl.BlockSpec((1, CIN), lambda i: (0, 0)),
            pl.BlockSpec((1, CIN), lambda i: (0, 0)),
            pl.BlockSpec((1, _W), lambda i: (0, 0)),
        ],
        out_specs=[
            pl.BlockSpec((_RB, _W), lambda i: (i, 0)),
            pl.BlockSpec((2, _W), lambda i: (0, 0)),
        ],
        out_shape=[
            jax.ShapeDtypeStruct((_PR, _W), jnp.float32),
            jax.ShapeDtypeStruct((2, _W), jnp.float32),
        ],
    )(xgp, beff, st1, w1bd, g0, be0, b1p)


def _layer3_body(y_ref, st_ref, w2bd_ref, g1_ref, be1_ref, b2p_ref,
                 m_ref, out_st_ref):
    i = pl.program_id(0)
    a, c = _norm_coefs(st_ref[...], g1_ref[...], be1_ref[...], CIN)
    h = jnp.maximum(a * y_ref[...] + c, 0.0)
    y = jnp.dot(h, w2bd_ref[...],
                preferred_element_type=jnp.float32) + b2p_ref[...]
    s = jnp.sum(y, axis=0)
    q = jnp.sum(y * y, axis=0)
    sq = jnp.stack([s, q], axis=0)
    m_ref[...] = jnp.max(y.reshape(_CB, _PK, 2, _COUT), axis=(1, 2))

    @pl.when(i == 0)
    def _():
        out_st_ref[...] = sq

    @pl.when(i != 0)
    def _():
        out_st_ref[...] += sq


def _layer3(y2, st2, w2bd, g1, be1, b2p):
    return pl.pallas_call(
        _layer3_body,
        grid=(_NB,),
        in_specs=[
            pl.BlockSpec((_RB, _W), lambda i: (i, 0)),
            pl.BlockSpec((2, _W), lambda i: (0, 0)),
            pl.BlockSpec((_W, 2 * _COUT), lambda i: (0, 0)),
            pl.BlockSpec((1, CIN), lambda i: (0, 0)),
            pl.BlockSpec((1, CIN), lambda i: (0, 0)),
            pl.BlockSpec((1, 2 * _COUT), lambda i: (0, 0)),
        ],
        out_specs=[
            pl.BlockSpec((_CB, _COUT), lambda i: (i, 0)),
            pl.BlockSpec((2, 2 * _COUT), lambda i: (0, 0)),
        ],
        out_shape=[
            jax.ShapeDtypeStruct((B * M, _COUT), jnp.float32),
            jax.ShapeDtypeStruct((2, 2 * _COUT), jnp.float32),
        ],
    )(y2, st2, w2bd, g1, be1, b2p)


def _final_body(m_ref, st_ref, g2_ref, be2_ref, out_ref):
    a, c = _norm_coefs(st_ref[...], g2_ref[...], be2_ref[...], _COUT)
    out_ref[...] = jnp.maximum(a[:, :_COUT] * m_ref[...] + c[:, :_COUT], 0.0)


def _final(m3, st3, g2, be2):
    return pl.pallas_call(
        _final_body,
        out_shape=jax.ShapeDtypeStruct((B * M, _COUT), jnp.float32),
    )(m3, st3, g2, be2)


@jax.jit
def kernel(points, features, W0, b0, g0, be0, W1, b1, g1, be1,
           W2, b2, g2, be2):
    pts_t = points.transpose(2, 0, 1)
    cent_t = _fps(pts_t)
    centroids = cent_t.transpose(1, 2, 0)

    v = _v_mat(points, features, W0[:, :3].T, W0[:, 3:].T)
    w0pt2 = jnp.concatenate([W0[:, :3].T, W0[:, :3].T], axis=1)
    b0p = jnp.concatenate([b0, b0]).reshape(1, _W)
    beff = _beff((centroids / RAD).reshape(B * M, 3), w0pt2, b0p)
    sqr = _sqr(cent_t.transpose(1, 0, 2), pts_t.transpose(1, 0, 2))

    xg = _sc_gather(sqr.reshape(B * M, N), v.reshape(B * N, CIN))
    xgp = xg.reshape(_PR, _W)

    w1bd = jnp.zeros((_W, _W), jnp.float32)
    w1bd = w1bd.at[:CIN, :CIN].set(W1.T).at[CIN:, CIN:].set(W1.T)
    w2bd = jnp.zeros((_W, 2 * _COUT), jnp.float32)
    w2bd = w2bd.at[:CIN, :_COUT].set(W2.T).at[CIN:, _COUT:].set(W2.T)
    b1p = jnp.concatenate([b1, b1]).reshape(1, _W)
    b2p = jnp.concatenate([b2, b2]).reshape(1, 2 * _COUT)

    st1 = _stats1(xgp, beff)
    y2, st2 = _layer2(xgp, beff, st1, w1bd, g0.reshape(1, CIN),
                      be0.reshape(1, CIN), b1p)
    m3, st3 = _layer3(y2, st2, w2bd, g1.reshape(1, CIN),
                      be1.reshape(1, CIN), b2p)
    g = _final(m3, st3, g2.reshape(1, _COUT), be2.reshape(1, _COUT))
    return (centroids, g.reshape(B, M, _COUT))
